# baseline (speedup 1.0000x reference)
import functools

import jax
import jax.numpy as jnp
import numpy as np
from jax import lax
from jax.experimental import pallas as pl
from jax.experimental.pallas import tpu as pltpu
from jax.experimental.pallas import tpu_sc as plsc

D_MODEL = 128
BATCH = 1024
SEQ = 200

_NC = 2
_NS = 16
_NW = _NC * _NS
_N_ROWS = BATCH * SEQ
_ROWS_PER_W = _N_ROWS // _NW
_CHUNK = 100
_NCHUNK = _ROWS_PER_W // _CHUNK
_NBUF = 4
_L = 16


def _pos_encoding():
    pos = np.arange(SEQ, dtype=np.float32)[:, None]
    i_even = np.arange(0, D_MODEL, 2, dtype=np.float32)[None, :]
    angles = pos / np.power(np.float32(10000.0), i_even / np.float32(D_MODEL))
    enc = np.zeros((SEQ, D_MODEL), dtype=np.float32)
    enc[:, 0::2] = np.sin(angles)
    enc[:, 1::2] = np.cos(angles)
    return jnp.asarray(enc)


def _body(idx_hbm, table_hbm, pos_hbm, out_hbm, idx_v, pos_v, rows_v, *sems):
    gsems = sems[:_NBUF]
    ssems = sems[_NBUF : 2 * _NBUF]
    psem = sems[2 * _NBUF]
    wid = lax.axis_index("s") * _NC + lax.axis_index("c")
    row0 = wid * _ROWS_PER_W

    nseq = _ROWS_PER_W // SEQ
    pltpu.sync_copy(idx_hbm.at[pl.ds(wid * nseq, nseq)], idx_v)

    def start_gather(c, b):
        sq = c // 2
        h = lax.rem(c, 2)
        pltpu.make_async_copy(
            table_hbm.at[idx_v.at[sq, h]], rows_v.at[b], gsems[b]
        ).start()

    def wait_gather(b):
        pltpu.make_async_copy(
            table_hbm.at[idx_v.at[0, 0]], rows_v.at[b], gsems[b]
        ).wait()

    def start_store(c, b):
        pltpu.make_async_copy(
            rows_v.at[b], out_hbm.at[pl.ds(row0 + c * _CHUNK, _CHUNK)], ssems[b]
        ).start()

    def wait_store(b):
        pltpu.make_async_copy(
            rows_v.at[b], out_hbm.at[pl.ds(row0, _CHUNK)], ssems[b]
        ).wait()

    def add_pos(c, b):
        pbase = lax.rem(c, 2) * _CHUNK

        def add_row(r, carry2):
            for j in range(D_MODEL // _L):
                v = pos_v[pbase + r, pl.ds(j * _L, _L)]
                plsc.addupdate(rows_v.at[b, r, pl.ds(j * _L, _L)], v)
            return carry2

        lax.fori_loop(0, _CHUNK, add_row, 0, unroll=4)

    for b in range(_NBUF):
        start_gather(b, b)
    pos_copy = pltpu.make_async_copy(pos_hbm, pos_v, psem)
    pos_copy.start()

    for b in range(_NBUF):
        nxt = b + 2
        b2 = nxt % _NBUF
        if nxt >= _NBUF:
            wait_store(b2)
            start_gather(nxt, b2)
        wait_gather(b)
        if b == 0:
            pos_copy.wait()
        add_pos(b, b)
        start_store(b, b)

    def round_body(g, carry):
        for b in range(_NBUF):
            c = g * _NBUF + b
            nxt = c + 2
            b2 = (b + 2) % _NBUF

            @pl.when(nxt < _NCHUNK)
            def _():
                wait_store(b2)
                start_gather(nxt, b2)

            wait_gather(b)
            add_pos(c, b)
            start_store(c, b)

        return carry

    lax.fori_loop(1, _NCHUNK // _NBUF, round_body, 0)

    for b in range(_NBUF):
        wait_store(b)


@jax.jit
def _emb(x, table):
    pos = _pos_encoding()
    xi = x.reshape(BATCH, 2, _CHUNK).astype(jnp.int32)
    run = pl.kernel(
        _body,
        mesh=plsc.VectorSubcoreMesh(core_axis_name="c", subcore_axis_name="s"),
        compiler_params=pltpu.CompilerParams(use_tc_tiling_on_sc=False),
        out_type=jax.ShapeDtypeStruct((_N_ROWS, D_MODEL), jnp.float32),
        scratch_types=[
            pltpu.VMEM((_ROWS_PER_W // SEQ, 2, _CHUNK), jnp.int32),
            pltpu.VMEM((SEQ, D_MODEL), jnp.float32),
            pltpu.VMEM((_NBUF, _CHUNK, D_MODEL), jnp.float32),
        ] + [pltpu.SemaphoreType.DMA] * (2 * _NBUF + 1),
    )
    out = run(xi, table, pos)
    return out.reshape(BATCH, SEQ, D_MODEL)


def kernel(x, table):
    return _emb(x, table)


# speedup vs baseline: 1.0209x; 1.0209x over previous
import functools

import jax
import jax.numpy as jnp
import numpy as np
from jax import lax
from jax.experimental import pallas as pl
from jax.experimental.pallas import tpu as pltpu
from jax.experimental.pallas import tpu_sc as plsc

D_MODEL = 128
BATCH = 1024
SEQ = 200

_NC = 2
_NS = 16
_NW = _NC * _NS
_N_ROWS = BATCH * SEQ
_ROWS_PER_W = _N_ROWS // _NW
_CHUNK = 100
_NCHUNK = _ROWS_PER_W // _CHUNK
_NBUF = 4
_L = 16


def _pos_encoding():
    pos = np.arange(SEQ, dtype=np.float32)[:, None]
    i_even = np.arange(0, D_MODEL, 2, dtype=np.float32)[None, :]
    angles = pos / np.power(np.float32(10000.0), i_even / np.float32(D_MODEL))
    enc = np.zeros((SEQ, D_MODEL), dtype=np.float32)
    enc[:, 0::2] = np.sin(angles)
    enc[:, 1::2] = np.cos(angles)
    return jnp.asarray(enc)


def _body(idx_hbm, table_hbm, pos_hbm, out_hbm, idx_v, pos_v, rows_v, *sems):
    gsems = sems[:_NBUF]
    ssems = sems[_NBUF : 2 * _NBUF]
    psem = sems[2 * _NBUF]
    wid = lax.axis_index("s") * _NC + lax.axis_index("c")
    row0 = wid * _ROWS_PER_W

    pltpu.sync_copy(idx_hbm.at[pl.ds(wid * _NCHUNK, _NCHUNK)], idx_v)

    def start_gather(c, b):
        pltpu.make_async_copy(
            table_hbm.at[idx_v.at[c]], rows_v.at[b], gsems[b]
        ).start()

    def wait_gather(b):
        pltpu.make_async_copy(
            table_hbm.at[idx_v.at[0]], rows_v.at[b], gsems[b]
        ).wait()

    def start_store(c, b):
        pltpu.make_async_copy(
            rows_v.at[b], out_hbm.at[pl.ds(row0 + c * _CHUNK, _CHUNK)], ssems[b]
        ).start()

    def wait_store(b):
        pltpu.make_async_copy(
            rows_v.at[b], out_hbm.at[pl.ds(row0, _CHUNK)], ssems[b]
        ).wait()

    def add_pos(c, b):
        pbase = lax.rem(c, 2) * _CHUNK

        def add_row(r, carry2):
            for j in range(D_MODEL // _L):
                v = pos_v[pbase + r, pl.ds(j * _L, _L)]
                plsc.addupdate(rows_v.at[b, r, pl.ds(j * _L, _L)], v)
            return carry2

        lax.fori_loop(0, _CHUNK, add_row, 0, unroll=4)

    for b in range(_NBUF):
        start_gather(b, b)
    pos_copy = pltpu.make_async_copy(pos_hbm, pos_v, psem)
    pos_copy.start()

    for b in range(_NBUF):
        nxt = b + 2
        b2 = nxt % _NBUF
        if nxt >= _NBUF:
            wait_store(b2)
            start_gather(nxt, b2)
        wait_gather(b)
        if b == 0:
            pos_copy.wait()
        add_pos(b, b)
        start_store(b, b)

    def round_body(g, carry):
        for b in range(_NBUF):
            c = g * _NBUF + b
            nxt = c + 2
            b2 = (b + 2) % _NBUF

            @pl.when(nxt < _NCHUNK)
            def _():
                wait_store(b2)
                start_gather(nxt, b2)

            wait_gather(b)
            add_pos(c, b)
            start_store(c, b)

        return carry

    lax.fori_loop(1, _NCHUNK // _NBUF, round_body, 0)

    for b in range(_NBUF):
        wait_store(b)


@jax.jit
def _emb(x, table):
    pos = _pos_encoding()
    xi = x.reshape(_N_ROWS // _CHUNK, _CHUNK).astype(jnp.int32)
    run = pl.kernel(
        _body,
        mesh=plsc.VectorSubcoreMesh(core_axis_name="c", subcore_axis_name="s"),
        compiler_params=pltpu.CompilerParams(use_tc_tiling_on_sc=False),
        out_type=jax.ShapeDtypeStruct((_N_ROWS, D_MODEL), jnp.float32),
        scratch_types=[
            pltpu.VMEM((_NCHUNK, _CHUNK), jnp.int32),
            pltpu.VMEM((SEQ, D_MODEL), jnp.float32),
            pltpu.VMEM((_NBUF, _CHUNK, D_MODEL), jnp.float32),
        ] + [pltpu.SemaphoreType.DMA] * (2 * _NBUF + 1),
    )
    out = run(xi, table, pos)
    return out.reshape(BATCH, SEQ, D_MODEL)


def kernel(x, table):
    return _emb(x, table)
